# Initial kernel scaffold; baseline (speedup 1.0000x reference)
import jax
import jax.numpy as jnp
from jax.experimental import pallas as pl


def kernel(inputs, table):
    raise NotImplementedError("write your pallas kernel here")


# speedup vs baseline: 3.3772x; 3.3772x over previous
import functools

import jax
import jax.numpy as jnp
from jax import lax
from jax.experimental import pallas as pl
from jax.experimental.pallas import tpu as pltpu
from jax.experimental.pallas import tpu_sc as plsc

B, L, C = 4096, 200, 128
VOCAB = 262
N = B * L
NUM_CORES = 2
NUM_SUBCORES = 16
NW = NUM_CORES * NUM_SUBCORES
TPW = N // NW
T = 128
CHUNKS = TPW // T


def _sc_body(idx_hbm, table_hbm, tab_ev_hbm, tab_od_hbm, x_hbm, p_hbm,
             pooled_hbm, ev_v, od_v, pooled_v, idx_v, xrows_v, prows_v,
             sem_x, sem_p):
    c = lax.axis_index("c")
    s = lax.axis_index("s")
    wid = s * NUM_CORES + c

    @pl.when(s == 0)
    def _():
        pltpu.sync_copy(tab_ev_hbm, ev_v)
        pltpu.sync_copy(tab_od_hbm, od_v)

        def row_body(r, carry):
            for j in range(4):
                sl = pl.ds(j * 16, 16)
                pooled_v[r, sl] = jnp.maximum(ev_v[r, sl], od_v[r, sl])
            return carry

        lax.fori_loop(0, VOCAB, row_body, 0)
        pltpu.sync_copy(pooled_v, pooled_hbm)

    plsc.subcore_barrier()

    base_w = wid * TPW

    def chunk_body(i, carry):
        b = base_w + i * T
        pltpu.sync_copy(idx_hbm.at[pl.ds(b, T)], idx_v)
        cx = pltpu.async_copy(table_hbm.at[idx_v], xrows_v, sem_x)
        cp = pltpu.async_copy(pooled_hbm.at[idx_v], prows_v, sem_p)
        cx.wait()
        cp.wait()
        pltpu.sync_copy(xrows_v, x_hbm.at[pl.ds(b, T)])
        pltpu.sync_copy(prows_v, p_hbm.at[pl.ds(b, T)])
        return carry

    lax.fori_loop(0, CHUNKS, chunk_body, 0)


@jax.jit
def kernel(inputs, table):
    idx = inputs.reshape(N).astype(jnp.int32)
    table = table.astype(jnp.float32)
    tab_ev = table[:, 0::2]
    tab_od = table[:, 1::2]
    mesh = plsc.VectorSubcoreMesh(core_axis_name="c", subcore_axis_name="s")
    call = pl.kernel(
        _sc_body,
        mesh=mesh,
        compiler_params=pltpu.CompilerParams(use_tc_tiling_on_sc=False),
        out_type=[
            jax.ShapeDtypeStruct((N, C), jnp.float32),
            jax.ShapeDtypeStruct((N, C // 2), jnp.float32),
            jax.ShapeDtypeStruct((VOCAB, C // 2), jnp.float32),
        ],
        scratch_types=[
            pltpu.VMEM((VOCAB, C // 2), jnp.float32),
            pltpu.VMEM((VOCAB, C // 2), jnp.float32),
            pltpu.VMEM((VOCAB, C // 2), jnp.float32),
            pltpu.VMEM((T,), jnp.int32),
            pltpu.VMEM((T, C), jnp.float32),
            pltpu.VMEM((T, C // 2), jnp.float32),
            pltpu.SemaphoreType.DMA,
            pltpu.SemaphoreType.DMA,
        ],
    )
    x, p, _pooled = call(idx, table, tab_ev, tab_od)
    return (x.reshape(B, L, C), p.reshape(B, L, C // 2))


# speedup vs baseline: 3.4291x; 1.0154x over previous
import functools

import jax
import jax.numpy as jnp
from jax import lax
from jax.experimental import pallas as pl
from jax.experimental.pallas import tpu as pltpu
from jax.experimental.pallas import tpu_sc as plsc

B, L, C = 4096, 200, 128
VOCAB = 262
N = B * L
NUM_CORES = 2
NUM_SUBCORES = 16
NW = NUM_CORES * NUM_SUBCORES
TPW = N // NW
T = 128
CHUNKS = TPW // T
NBUF = 4
NG = CHUNKS // NBUF
VHALF = VOCAB // 2


def _sc_body(idx_hbm, table_hbm, tab_ev_hbm, tab_od_hbm, x_hbm, p_hbm,
             pooled_hbm, pa_v, pb_v, idx_v, xrows_v, prows_v,
             sem_i, sem_g, sem_o):
    c = lax.axis_index("c")
    s = lax.axis_index("s")
    wid = s * NUM_CORES + c

    @pl.when(s == 0)
    def _():
        for h in range(2):
            rows = pl.ds(h * VHALF, VHALF)
            pltpu.sync_copy(tab_ev_hbm.at[rows], pa_v)
            pltpu.sync_copy(tab_od_hbm.at[rows], pb_v)

            def row_body(r, carry):
                for j in range(4):
                    sl = pl.ds(j * 16, 16)
                    pa_v[r, sl] = jnp.maximum(pa_v[r, sl], pb_v[r, sl])
                return carry

            lax.fori_loop(0, VHALF, row_body, 0)
            pltpu.sync_copy(pa_v, pooled_hbm.at[rows])

    plsc.subcore_barrier()

    base_w = wid * TPW

    def idx_copy(i, k):
        return pltpu.make_async_copy(
            idx_hbm.at[pl.ds(base_w + i * T, T)], idx_v.at[k], sem_i.at[k])

    def gx_copy(k):
        return pltpu.make_async_copy(
            table_hbm.at[idx_v.at[k]], xrows_v.at[k], sem_g.at[k])

    def gp_copy(k):
        return pltpu.make_async_copy(
            pooled_hbm.at[idx_v.at[k]], prows_v.at[k], sem_g.at[k])

    def ox_copy(i, k):
        return pltpu.make_async_copy(
            xrows_v.at[k], x_hbm.at[pl.ds(base_w + i * T, T)], sem_o.at[k])

    def op_copy(i, k):
        return pltpu.make_async_copy(
            prows_v.at[k], p_hbm.at[pl.ds(base_w + i * T, T)], sem_o.at[k])

    for k in range(NBUF):
        idx_copy(k, k).start()
    for k in range(NBUF):
        idx_copy(k, k).wait()
        gx_copy(k).start()
        gp_copy(k).start()

    def ring_body(g, carry):
        for k in range(NBUF):
            i = g * NBUF + k
            gx_copy(k).wait()
            gp_copy(k).wait()
            ox_copy(i, k).start()
            op_copy(i, k).start()

            @pl.when(g < NG - 1)
            def _():
                j = i + NBUF
                idx_copy(j, k).start()
                ox_copy(i, k).wait()
                op_copy(i, k).wait()
                idx_copy(j, k).wait()
                gx_copy(k).start()
                gp_copy(k).start()
        return carry

    lax.fori_loop(0, NG, ring_body, 0)

    for k in range(NBUF):
        i = (NG - 1) * NBUF + k
        ox_copy(i, k).wait()
        op_copy(i, k).wait()


@jax.jit
def kernel(inputs, table):
    idx = inputs.reshape(N).astype(jnp.int32)
    table = table.astype(jnp.float32)
    tab_ev = table[:, 0::2]
    tab_od = table[:, 1::2]
    mesh = plsc.VectorSubcoreMesh(core_axis_name="c", subcore_axis_name="s")
    call = pl.kernel(
        _sc_body,
        mesh=mesh,
        compiler_params=pltpu.CompilerParams(use_tc_tiling_on_sc=False),
        out_type=[
            jax.ShapeDtypeStruct((N, C), jnp.float32),
            jax.ShapeDtypeStruct((N, C // 2), jnp.float32),
            jax.ShapeDtypeStruct((VOCAB, C // 2), jnp.float32),
        ],
        scratch_types=[
            pltpu.VMEM((VHALF, C // 2), jnp.float32),
            pltpu.VMEM((VHALF, C // 2), jnp.float32),
            pltpu.VMEM((NBUF, T), jnp.int32),
            pltpu.VMEM((NBUF, T, C), jnp.float32),
            pltpu.VMEM((NBUF, T, C // 2), jnp.float32),
            pltpu.SemaphoreType.DMA((NBUF,)),
            pltpu.SemaphoreType.DMA((NBUF,)),
            pltpu.SemaphoreType.DMA((NBUF,)),
        ],
    )
    x, p, _pooled = call(idx, table, tab_ev, tab_od)
    return (x.reshape(B, L, C), p.reshape(B, L, C // 2))
